# Initial kernel scaffold; baseline (speedup 1.0000x reference)
import jax
import jax.numpy as jnp
from jax.experimental import pallas as pl


def kernel(u, edge_index, batch, W1, att_src1, att_dst1, b1, W2, att_src2, att_dst2, b2):
    raise NotImplementedError("write your pallas kernel here")


# speedup vs baseline: 29.2514x; 29.2514x over previous
import functools

import jax
import jax.numpy as jnp
from jax import lax
from jax.experimental import pallas as pl
from jax.experimental.pallas import tpu as pltpu
from jax.experimental.pallas import tpu_sc as plsc

N = 10000
E = 320000
D = 128
G = 64

NC = 2
NS = 16
NW = NC * NS
EPT = E // NW
CH = 80
NCH = EPT // CH
NPAD = 10112
RPS = NPAD // NS

_SELU_ALPHA = 1.6732632423543772848170429916717
_SELU_SCALE = 1.0507009873554804934193349852946


def _selu(x):
    safe = jnp.minimum(x, 0.0)
    return _SELU_SCALE * jnp.where(
        x > 0, x, _SELU_ALPHA * (jnp.exp(safe) - 1.0))


def _sc_layer_body(src_h, dst_h, asrc_h, adst_h, h_h, z2_h, z1_h,
                   acc_o, den_o,
                   src_v, dst_v, as_c, ad_c, w_c, rows_v, den_b,
                   acc_sh, den_sh, sem0, sem1, sem2):
    cid = lax.axis_index("c")
    sid = lax.axis_index("s")
    wid = cid * NS + sid
    r0 = sid * RPS

    pltpu.sync_copy(z2_h.at[pl.ds(r0, RPS)], acc_sh.at[pl.ds(r0, RPS)])
    pltpu.sync_copy(z1_h.at[pl.ds(r0, RPS)], den_b)
    pltpu.sync_copy(den_b, den_sh.at[pl.ds(r0, RPS)])

    pltpu.sync_copy(src_h.at[wid], src_v)
    pltpu.sync_copy(dst_h.at[wid], dst_v)
    plsc.subcore_barrier()

    def cbody(c, carry):
        src_row = src_v.at[c]
        dst_row = dst_v.at[c]
        d0 = pltpu.async_copy(asrc_h.at[src_row], as_c, sem0)
        d1 = pltpu.async_copy(adst_h.at[dst_row], ad_c, sem1)
        d2 = pltpu.async_copy(h_h.at[src_row], rows_v, sem2)
        d0.wait()
        d1.wait()
        for j in range(CH // 16):
            a = as_c[pl.ds(j * 16, 16)] + ad_c[pl.ds(j * 16, 16)]
            a = jnp.maximum(a, 0.2 * a)
            w_c[pl.ds(j * 16, 16)] = jnp.exp(a)
        d2.wait()

        def ebody(e, ecarry):
            wb = plsc.load_gather(w_c, [jnp.full((16,), e, jnp.int32)])
            for j in range(D // 16):
                rows_v[e, pl.ds(j * 16, 16)] = rows_v[e, pl.ds(j * 16, 16)] * wb
            return ecarry

        lax.fori_loop(0, CH, ebody, 0)
        pltpu.sync_copy(rows_v, acc_sh.at[dst_row], add=True)
        pltpu.sync_copy(w_c, den_sh.at[dst_row], add=True)
        return carry

    lax.fori_loop(0, NCH, cbody, 0)
    plsc.subcore_barrier()

    pltpu.sync_copy(acc_sh.at[pl.ds(r0, RPS)], acc_o.at[cid, pl.ds(r0, RPS)])
    pltpu.sync_copy(den_sh.at[pl.ds(r0, RPS)], den_b)
    pltpu.sync_copy(den_b, den_o.at[pl.ds(cid * NPAD + r0, RPS)])


_sc_layer = pl.kernel(
    _sc_layer_body,
    out_type=[
        jax.ShapeDtypeStruct((NC, NPAD, D), jnp.float32),
        jax.ShapeDtypeStruct((NC * NPAD,), jnp.float32),
    ],
    mesh=plsc.VectorSubcoreMesh(core_axis_name="c", subcore_axis_name="s"),
    compiler_params=pltpu.CompilerParams(needs_layout_passes=False),
    scratch_types=[
        pltpu.VMEM((NCH, CH), jnp.int32),
        pltpu.VMEM((NCH, CH), jnp.int32),
        pltpu.VMEM((CH,), jnp.float32),
        pltpu.VMEM((CH,), jnp.float32),
        pltpu.VMEM((CH,), jnp.float32),
        pltpu.VMEM((CH, D), jnp.float32),
        pltpu.VMEM((RPS,), jnp.float32),
        pltpu.VMEM_SHARED((NPAD, D), jnp.float32),
        pltpu.VMEM_SHARED((NPAD,), jnp.float32),
        pltpu.SemaphoreType.DMA,
        pltpu.SemaphoreType.DMA,
        pltpu.SemaphoreType.DMA,
    ],
)


def _tc_pre_body(u_ref, w_ref, asw_ref, adw_ref, h_ref, asrc_ref, adst_ref,
                 wself_ref):
    h = jnp.dot(u_ref[...], w_ref[...], preferred_element_type=jnp.float32)
    h_ref[...] = h
    asrc = jnp.dot(h, asw_ref[...], preferred_element_type=jnp.float32)
    adst = jnp.dot(h, adw_ref[...], preferred_element_type=jnp.float32)
    asrc_ref[...] = asrc
    adst_ref[...] = adst
    a = asrc + adst
    wself_ref[...] = jnp.exp(jnp.maximum(a, 0.2 * a))


_tc_pre = pl.pallas_call(
    _tc_pre_body,
    out_shape=[
        jax.ShapeDtypeStruct((N, D), jnp.float32),
        jax.ShapeDtypeStruct((N, 1), jnp.float32),
        jax.ShapeDtypeStruct((N, 1), jnp.float32),
        jax.ShapeDtypeStruct((N, 1), jnp.float32),
    ],
)


def _tc_mid_body(acc_ref, den_ref, h_ref, wself_ref, b_ref, w_ref, asw_ref,
                 adw_ref, h2_ref, asrc_ref, adst_ref, wself2_ref):
    num = acc_ref[0, :N, :] + acc_ref[1, :N, :] + wself_ref[...] * h_ref[...]
    den = (den_ref[0:1, :N] + den_ref[1:2, :N]).reshape(N, 1) + \
        wself_ref[...] + 1e-16
    x = _selu(num / den + b_ref[...])
    h2 = jnp.dot(x, w_ref[...], preferred_element_type=jnp.float32)
    h2_ref[...] = h2
    asrc = jnp.dot(h2, asw_ref[...], preferred_element_type=jnp.float32)
    adst = jnp.dot(h2, adw_ref[...], preferred_element_type=jnp.float32)
    asrc_ref[...] = asrc
    adst_ref[...] = adst
    a = asrc + adst
    wself2_ref[...] = jnp.exp(jnp.maximum(a, 0.2 * a))


_tc_mid = pl.pallas_call(
    _tc_mid_body,
    out_shape=[
        jax.ShapeDtypeStruct((N, D), jnp.float32),
        jax.ShapeDtypeStruct((N, 1), jnp.float32),
        jax.ShapeDtypeStruct((N, 1), jnp.float32),
        jax.ShapeDtypeStruct((N, 1), jnp.float32),
    ],
)


def _tc_post_body(acc_ref, den_ref, h_ref, wself_ref, b_ref, batch_ref,
                  out_ref):
    num = acc_ref[0, :N, :] + acc_ref[1, :N, :] + wself_ref[...] * h_ref[...]
    den = (den_ref[0:1, :N] + den_ref[1:2, :N]).reshape(N, 1) + \
        wself_ref[...] + 1e-16
    y = _selu(num / den + b_ref[...])
    gids = lax.broadcasted_iota(jnp.int32, (G, N), 0)
    onehot = (gids == batch_ref[...]).astype(jnp.float32)
    sums = jnp.dot(onehot, y, preferred_element_type=jnp.float32)
    cnts = jnp.sum(onehot, axis=1, keepdims=True)
    out_ref[...] = sums / jnp.clip(cnts, 1.0, None)


_tc_post = pl.pallas_call(
    _tc_post_body,
    out_shape=jax.ShapeDtypeStruct((G, D), jnp.float32),
)


def kernel(u, edge_index, batch, W1, att_src1, att_dst1, b1,
           W2, att_src2, att_dst2, b2):
    ei = jnp.asarray(edge_index, jnp.int32)
    src3 = ei[0].reshape(NW, NCH, CH)
    dst3 = ei[1].reshape(NW, NCH, CH)
    batch2 = jnp.asarray(batch, jnp.int32).reshape(1, N)
    z2 = jnp.zeros((NPAD, D), jnp.float32)
    z1 = jnp.zeros((NPAD,), jnp.float32)

    h1, asrc1, adst1, wself1 = _tc_pre(
        u, W1, att_src1.reshape(D, 1), att_dst1.reshape(D, 1))
    acc1, den1 = _sc_layer(src3, dst3, asrc1.reshape(N), adst1.reshape(N),
                           h1, z2, z1)
    h2, asrc2, adst2, wself2 = _tc_mid(
        acc1, den1.reshape(NC, NPAD), h1, wself1, b1.reshape(1, D), W2,
        att_src2.reshape(D, 1), att_dst2.reshape(D, 1))
    acc2, den2 = _sc_layer(src3, dst3, asrc2.reshape(N), adst2.reshape(N),
                           h2, z2, z1)
    return _tc_post(acc2, den2.reshape(NC, NPAD), h2, wself2,
                    b2.reshape(1, D), batch2)


# speedup vs baseline: 54.7666x; 1.8723x over previous
import functools

import jax
import jax.numpy as jnp
from jax import lax
from jax.experimental import pallas as pl
from jax.experimental.pallas import tpu as pltpu
from jax.experimental.pallas import tpu_sc as plsc

N = 10000
E = 320000
D = 128
G = 64

NC = 2
NS = 16
NW = NC * NS
EPT = E // NW
CH = 80
NCH = EPT // CH
NPAD = 10112
RPS = NPAD // NS

_SELU_ALPHA = 1.6732632423543772848170429916717
_SELU_SCALE = 1.0507009873554804934193349852946


def _selu(x):
    safe = jnp.minimum(x, 0.0)
    return _SELU_SCALE * jnp.where(
        x > 0, x, _SELU_ALPHA * (jnp.exp(safe) - 1.0))


def _sc_layer_body(src_h, dst_h, asrc_h, adst_h, h_h, z2_h, z1_h,
                   acc_o, den_o,
                   src_c0, src_c1, src_c2, dst_c0, dst_c1, dst_c2,
                   dst_s0, dst_s1, dst_s2, as_c0, as_c1, as_c2,
                   ad_c0, ad_c1, ad_c2, w_c0, w_c1, w_c2,
                   rows0, rows1, rows2, den_b,
                   acc_sh, den_sh,
                   g1s0, g1s1, g1s2, g2s0, g2s1, g2s2, ss0, ss1, ss2):
    src_c = [src_c0, src_c1, src_c2]
    dst_c = [dst_c0, dst_c1, dst_c2]
    dst_s = [dst_s0, dst_s1, dst_s2]
    as_c = [as_c0, as_c1, as_c2]
    ad_c = [ad_c0, ad_c1, ad_c2]
    w_c = [w_c0, w_c1, w_c2]
    rows = [rows0, rows1, rows2]
    g1s = [g1s0, g1s1, g1s2]
    g2s = [g2s0, g2s1, g2s2]
    ss = [ss0, ss1, ss2]

    cid = lax.axis_index("c")
    sid = lax.axis_index("s")
    wid = cid * NS + sid
    r0 = sid * RPS

    pltpu.sync_copy(z2_h.at[pl.ds(r0, RPS)], acc_sh.at[pl.ds(r0, RPS)])
    pltpu.sync_copy(z1_h.at[pl.ds(r0, RPS)], den_b)
    pltpu.sync_copy(den_b, den_sh.at[pl.ds(r0, RPS)])
    plsc.subcore_barrier()

    def g1_descs(c, s):
        o = (wid * NCH + c) * CH
        return (pltpu.make_async_copy(src_h.at[pl.ds(o, CH)], src_c[s], g1s[s]),
                pltpu.make_async_copy(dst_h.at[pl.ds(o, CH)], dst_c[s], g1s[s]))

    def g2_descs(s):
        return (pltpu.make_async_copy(asrc_h.at[src_c[s]], as_c[s], g2s[s]),
                pltpu.make_async_copy(adst_h.at[dst_c[s]], ad_c[s], g2s[s]),
                pltpu.make_async_copy(h_h.at[src_c[s]], rows[s], g2s[s]))

    def s_descs(s):
        return (pltpu.make_async_copy(rows[s], acc_sh.at[dst_s[s]], ss[s]),
                pltpu.make_async_copy(w_c[s], den_sh.at[dst_s[s]], ss[s]))

    def issue_g1(c, s):
        o = (wid * NCH + c) * CH
        pltpu.async_copy(src_h.at[pl.ds(o, CH)], src_c[s], g1s[s])
        pltpu.async_copy(dst_h.at[pl.ds(o, CH)], dst_c[s], g1s[s])

    def issue_g2(s):
        pltpu.async_copy(asrc_h.at[src_c[s]], as_c[s], g2s[s])
        pltpu.async_copy(adst_h.at[dst_c[s]], ad_c[s], g2s[s])
        pltpu.async_copy(h_h.at[src_c[s]], rows[s], g2s[s])

    def issue_s(s):
        pltpu.async_copy(rows[s], acc_sh.at[dst_s[s]], ss[s], add=True)
        pltpu.async_copy(w_c[s], den_sh.at[dst_s[s]], ss[s], add=True)

    def wait_all(descs):
        for d in descs:
            d.wait()

    def process(s):
        for j in range(CH // 16):
            dst_s[s][pl.ds(j * 16, 16)] = dst_c[s][pl.ds(j * 16, 16)]
            a = as_c[s][pl.ds(j * 16, 16)] + ad_c[s][pl.ds(j * 16, 16)]
            a = jnp.maximum(a, 0.2 * a)
            w_c[s][pl.ds(j * 16, 16)] = jnp.exp(a)

        def ebody(e4, ecarry):
            for i in range(4):
                e = e4 * 4 + i
                wb = plsc.load_gather(w_c[s],
                                      [jnp.full((16,), e, jnp.int32)])
                for j in range(D // 16):
                    rows[s][e, pl.ds(j * 16, 16)] = (
                        rows[s][e, pl.ds(j * 16, 16)] * wb)
            return ecarry

        lax.fori_loop(0, CH // 4, ebody, 0)

    issue_g1(0, 0)
    issue_g1(1, 1)
    wait_all(g1_descs(0, 0))
    issue_g2(0)

    def cbody(cc, carry):
        for k in range(3):
            c = cc * 3 + k
            s, s1, s2 = k, (k + 1) % 3, (k + 2) % 3

            @pl.when(c >= 2)
            def _():
                wait_all(s_descs(s1))
            wait_all(g1_descs(c + 1, s1))
            issue_g2(s1)
            issue_g1(c + 2, s2)
            wait_all(g2_descs(s))
            process(s)
            issue_s(s)
        return carry

    lax.fori_loop(0, (NCH - 2) // 3, cbody, 0)

    wait_all(s_descs(1))
    wait_all(g1_descs(NCH - 1, 1))
    issue_g2(1)
    wait_all(g2_descs(0))
    process(0)
    issue_s(0)
    wait_all(s_descs(2))
    wait_all(g2_descs(1))
    process(1)
    issue_s(1)
    wait_all(s_descs(0))
    wait_all(s_descs(1))
    plsc.subcore_barrier()

    pltpu.sync_copy(acc_sh.at[pl.ds(r0, RPS)], acc_o.at[cid, pl.ds(r0, RPS)])
    pltpu.sync_copy(den_sh.at[pl.ds(r0, RPS)], den_b)
    pltpu.sync_copy(den_b, den_o.at[pl.ds(cid * NPAD + r0, RPS)])


_sc_layer = pl.kernel(
    _sc_layer_body,
    out_type=[
        jax.ShapeDtypeStruct((NC, NPAD, D), jnp.float32),
        jax.ShapeDtypeStruct((NC * NPAD,), jnp.float32),
    ],
    mesh=plsc.VectorSubcoreMesh(core_axis_name="c", subcore_axis_name="s"),
    compiler_params=pltpu.CompilerParams(needs_layout_passes=False),
    scratch_types=(
        [pltpu.VMEM((CH,), jnp.int32) for _ in range(9)]
        + [pltpu.VMEM((CH,), jnp.float32) for _ in range(9)]
        + [pltpu.VMEM((CH, D), jnp.float32) for _ in range(3)]
        + [
            pltpu.VMEM((RPS,), jnp.float32),
            pltpu.VMEM_SHARED((NPAD, D), jnp.float32),
            pltpu.VMEM_SHARED((NPAD,), jnp.float32),
        ]
        + [pltpu.SemaphoreType.DMA for _ in range(9)]
    ),
)


def _tc_pre_body(u_ref, w_ref, asw_ref, adw_ref, h_ref, asrc_ref, adst_ref,
                 wself_ref):
    h = jnp.dot(u_ref[...], w_ref[...], preferred_element_type=jnp.float32)
    h_ref[...] = h
    asrc = jnp.dot(h, asw_ref[...], preferred_element_type=jnp.float32)
    adst = jnp.dot(h, adw_ref[...], preferred_element_type=jnp.float32)
    asrc_ref[...] = asrc
    adst_ref[...] = adst
    a = asrc + adst
    wself_ref[...] = jnp.exp(jnp.maximum(a, 0.2 * a))


_tc_pre = pl.pallas_call(
    _tc_pre_body,
    out_shape=[
        jax.ShapeDtypeStruct((N, D), jnp.float32),
        jax.ShapeDtypeStruct((N, 1), jnp.float32),
        jax.ShapeDtypeStruct((N, 1), jnp.float32),
        jax.ShapeDtypeStruct((N, 1), jnp.float32),
    ],
)


def _tc_mid_body(acc_ref, den_ref, h_ref, wself_ref, b_ref, w_ref, asw_ref,
                 adw_ref, h2_ref, asrc_ref, adst_ref, wself2_ref):
    num = acc_ref[0, :N, :] + acc_ref[1, :N, :] + wself_ref[...] * h_ref[...]
    den = (den_ref[0:1, :N] + den_ref[1:2, :N]).reshape(N, 1) + \
        wself_ref[...] + 1e-16
    x = _selu(num / den + b_ref[...])
    h2 = jnp.dot(x, w_ref[...], preferred_element_type=jnp.float32)
    h2_ref[...] = h2
    asrc = jnp.dot(h2, asw_ref[...], preferred_element_type=jnp.float32)
    adst = jnp.dot(h2, adw_ref[...], preferred_element_type=jnp.float32)
    asrc_ref[...] = asrc
    adst_ref[...] = adst
    a = asrc + adst
    wself2_ref[...] = jnp.exp(jnp.maximum(a, 0.2 * a))


_tc_mid = pl.pallas_call(
    _tc_mid_body,
    out_shape=[
        jax.ShapeDtypeStruct((N, D), jnp.float32),
        jax.ShapeDtypeStruct((N, 1), jnp.float32),
        jax.ShapeDtypeStruct((N, 1), jnp.float32),
        jax.ShapeDtypeStruct((N, 1), jnp.float32),
    ],
)


def _tc_post_body(acc_ref, den_ref, h_ref, wself_ref, b_ref, batch_ref,
                  out_ref):
    num = acc_ref[0, :N, :] + acc_ref[1, :N, :] + wself_ref[...] * h_ref[...]
    den = (den_ref[0:1, :N] + den_ref[1:2, :N]).reshape(N, 1) + \
        wself_ref[...] + 1e-16
    y = _selu(num / den + b_ref[...])
    gids = lax.broadcasted_iota(jnp.int32, (G, N), 0)
    onehot = (gids == batch_ref[...]).astype(jnp.float32)
    sums = jnp.dot(onehot, y, preferred_element_type=jnp.float32)
    cnts = jnp.sum(onehot, axis=1, keepdims=True)
    out_ref[...] = sums / jnp.clip(cnts, 1.0, None)


_tc_post = pl.pallas_call(
    _tc_post_body,
    out_shape=jax.ShapeDtypeStruct((G, D), jnp.float32),
)


def kernel(u, edge_index, batch, W1, att_src1, att_dst1, b1,
           W2, att_src2, att_dst2, b2):
    ei = jnp.asarray(edge_index, jnp.int32)
    src3 = ei[0]
    dst3 = ei[1]
    batch2 = jnp.asarray(batch, jnp.int32).reshape(1, N)
    z2 = jnp.zeros((NPAD, D), jnp.float32)
    z1 = jnp.zeros((NPAD,), jnp.float32)

    h1, asrc1, adst1, wself1 = _tc_pre(
        u, W1, att_src1.reshape(D, 1), att_dst1.reshape(D, 1))
    acc1, den1 = _sc_layer(src3, dst3, asrc1.reshape(N), adst1.reshape(N),
                           h1, z2, z1)
    h2, asrc2, adst2, wself2 = _tc_mid(
        acc1, den1.reshape(NC, NPAD), h1, wself1, b1.reshape(1, D), W2,
        att_src2.reshape(D, 1), att_dst2.reshape(D, 1))
    acc2, den2 = _sc_layer(src3, dst3, asrc2.reshape(N), adst2.reshape(N),
                           h2, z2, z1)
    return _tc_post(acc2, den2.reshape(NC, NPAD), h2, wself2,
                    b2.reshape(1, D), batch2)


# speedup vs baseline: 58.5827x; 1.0697x over previous
import functools

import jax
import jax.numpy as jnp
from jax import lax
from jax.experimental import pallas as pl
from jax.experimental.pallas import tpu as pltpu
from jax.experimental.pallas import tpu_sc as plsc

N = 10000
E = 320000
D = 128
G = 64

NC = 2
NS = 16
NW = NC * NS
EPT = E // NW
CH = 80
NCH = EPT // CH
NPAD = 10112
RPS = NPAD // NS

_SELU_ALPHA = 1.6732632423543772848170429916717
_SELU_SCALE = 1.0507009873554804934193349852946


def _selu(x):
    safe = jnp.minimum(x, 0.0)
    return _SELU_SCALE * jnp.where(
        x > 0, x, _SELU_ALPHA * (jnp.exp(safe) - 1.0))


def _sc_layer_body(src_h, dst_h, asrc_h, adst_h, h_h, z2_h, z1_h,
                   acc_o, den_o,
                   src_c0, src_c1, src_c2, dst_c0, dst_c1, dst_c2,
                   dst_s0, dst_s1, dst_s2, as_c0, as_c1, as_c2,
                   ad_c0, ad_c1, ad_c2, w_c0, w_c1, w_c2,
                   rows0, rows1, rows2, den_b,
                   acc_sh, den_sh,
                   g1s0, g1s1, g1s2, g2s0, g2s1, g2s2, ss0, ss1, ss2):
    src_c = [src_c0, src_c1, src_c2]
    dst_c = [dst_c0, dst_c1, dst_c2]
    dst_s = [dst_s0, dst_s1, dst_s2]
    as_c = [as_c0, as_c1, as_c2]
    ad_c = [ad_c0, ad_c1, ad_c2]
    w_c = [w_c0, w_c1, w_c2]
    rows = [rows0, rows1, rows2]
    g1s = [g1s0, g1s1, g1s2]
    g2s = [g2s0, g2s1, g2s2]
    ss = [ss0, ss1, ss2]

    cid = lax.axis_index("c")
    sid = lax.axis_index("s")
    wid = cid * NS + sid
    r0 = sid * RPS

    pltpu.sync_copy(z2_h.at[pl.ds(r0, RPS)], acc_sh.at[pl.ds(r0, RPS)])
    pltpu.sync_copy(z1_h.at[pl.ds(r0, RPS)], den_b)
    pltpu.sync_copy(den_b, den_sh.at[pl.ds(r0, RPS)])
    plsc.subcore_barrier()

    def g1_descs(c, s):
        o = (wid * NCH + c) * CH
        return (pltpu.make_async_copy(src_h.at[pl.ds(o, CH)], src_c[s], g1s[s]),
                pltpu.make_async_copy(dst_h.at[pl.ds(o, CH)], dst_c[s], g1s[s]))

    def g2_descs(s):
        return (pltpu.make_async_copy(asrc_h.at[src_c[s]], as_c[s], g2s[s]),
                pltpu.make_async_copy(adst_h.at[dst_c[s]], ad_c[s], g2s[s]),
                pltpu.make_async_copy(h_h.at[src_c[s]], rows[s], g2s[s]))

    def s_descs(s):
        return (pltpu.make_async_copy(rows[s], acc_sh.at[dst_s[s]], ss[s]),
                pltpu.make_async_copy(w_c[s], den_sh.at[dst_s[s]], ss[s]))

    def issue_g1(c, s):
        o = (wid * NCH + c) * CH
        pltpu.async_copy(src_h.at[pl.ds(o, CH)], src_c[s], g1s[s])
        pltpu.async_copy(dst_h.at[pl.ds(o, CH)], dst_c[s], g1s[s])

    def issue_g2(s):
        pltpu.async_copy(asrc_h.at[src_c[s]], as_c[s], g2s[s])
        pltpu.async_copy(adst_h.at[dst_c[s]], ad_c[s], g2s[s])
        pltpu.async_copy(h_h.at[src_c[s]], rows[s], g2s[s])

    def issue_s(s):
        pltpu.async_copy(rows[s], acc_sh.at[dst_s[s]], ss[s], add=True)
        pltpu.async_copy(w_c[s], den_sh.at[dst_s[s]], ss[s], add=True)

    def wait_all(descs):
        for d in descs:
            d.wait()

    def process(s):
        for j in range(CH // 16):
            dst_s[s][pl.ds(j * 16, 16)] = dst_c[s][pl.ds(j * 16, 16)]
            a = as_c[s][pl.ds(j * 16, 16)] + ad_c[s][pl.ds(j * 16, 16)]
            a = jnp.maximum(a, 0.2 * a)
            w_c[s][pl.ds(j * 16, 16)] = jnp.exp(a)

        @plsc.parallel_loop(0, CH, step=4, unroll=5)
        def _scale(e4):
            for i in range(4):
                e = e4 + i
                wb = plsc.load_gather(w_c[s],
                                      [jnp.full((16,), e, jnp.int32)])
                for j in range(D // 16):
                    rows[s][e, pl.ds(j * 16, 16)] = (
                        rows[s][e, pl.ds(j * 16, 16)] * wb)

    issue_g1(0, 0)
    issue_g1(1, 1)
    wait_all(g1_descs(0, 0))
    issue_g2(0)

    def cbody(cc, carry):
        for k in range(3):
            c = cc * 3 + k
            s, s1, s2 = k, (k + 1) % 3, (k + 2) % 3

            @pl.when(c >= 2)
            def _():
                wait_all(s_descs(s1))
            wait_all(g1_descs(c + 1, s1))
            issue_g2(s1)
            issue_g1(c + 2, s2)
            wait_all(g2_descs(s))
            process(s)
            issue_s(s)
        return carry

    lax.fori_loop(0, (NCH - 2) // 3, cbody, 0)

    wait_all(s_descs(1))
    wait_all(g1_descs(NCH - 1, 1))
    issue_g2(1)
    wait_all(g2_descs(0))
    process(0)
    issue_s(0)
    wait_all(s_descs(2))
    wait_all(g2_descs(1))
    process(1)
    issue_s(1)
    wait_all(s_descs(0))
    wait_all(s_descs(1))
    plsc.subcore_barrier()

    pltpu.sync_copy(acc_sh.at[pl.ds(r0, RPS)], acc_o.at[cid, pl.ds(r0, RPS)])
    pltpu.sync_copy(den_sh.at[pl.ds(r0, RPS)], den_b)
    pltpu.sync_copy(den_b, den_o.at[pl.ds(cid * NPAD + r0, RPS)])


_sc_layer = pl.kernel(
    _sc_layer_body,
    out_type=[
        jax.ShapeDtypeStruct((NC, NPAD, D), jnp.float32),
        jax.ShapeDtypeStruct((NC * NPAD,), jnp.float32),
    ],
    mesh=plsc.VectorSubcoreMesh(core_axis_name="c", subcore_axis_name="s"),
    compiler_params=pltpu.CompilerParams(needs_layout_passes=False),
    scratch_types=(
        [pltpu.VMEM((CH,), jnp.int32) for _ in range(9)]
        + [pltpu.VMEM((CH,), jnp.float32) for _ in range(9)]
        + [pltpu.VMEM((CH, D), jnp.float32) for _ in range(3)]
        + [
            pltpu.VMEM((RPS,), jnp.float32),
            pltpu.VMEM_SHARED((NPAD, D), jnp.float32),
            pltpu.VMEM_SHARED((NPAD,), jnp.float32),
        ]
        + [pltpu.SemaphoreType.DMA for _ in range(9)]
    ),
)


def _tc_pre_body(u_ref, w_ref, asw_ref, adw_ref, h_ref, asrc_ref, adst_ref,
                 wself_ref):
    h = jnp.dot(u_ref[...], w_ref[...], preferred_element_type=jnp.float32)
    h_ref[...] = h
    asrc = jnp.dot(h, asw_ref[...], preferred_element_type=jnp.float32)
    adst = jnp.dot(h, adw_ref[...], preferred_element_type=jnp.float32)
    asrc_ref[...] = asrc
    adst_ref[...] = adst
    a = asrc + adst
    wself_ref[...] = jnp.exp(jnp.maximum(a, 0.2 * a))


_tc_pre = pl.pallas_call(
    _tc_pre_body,
    out_shape=[
        jax.ShapeDtypeStruct((N, D), jnp.float32),
        jax.ShapeDtypeStruct((N, 1), jnp.float32),
        jax.ShapeDtypeStruct((N, 1), jnp.float32),
        jax.ShapeDtypeStruct((N, 1), jnp.float32),
    ],
)


def _tc_mid_body(acc_ref, den_ref, h_ref, wself_ref, b_ref, w_ref, asw_ref,
                 adw_ref, h2_ref, asrc_ref, adst_ref, wself2_ref):
    num = acc_ref[0, :N, :] + acc_ref[1, :N, :] + wself_ref[...] * h_ref[...]
    den = (den_ref[0:1, :N] + den_ref[1:2, :N]).reshape(N, 1) + \
        wself_ref[...] + 1e-16
    x = _selu(num / den + b_ref[...])
    h2 = jnp.dot(x, w_ref[...], preferred_element_type=jnp.float32)
    h2_ref[...] = h2
    asrc = jnp.dot(h2, asw_ref[...], preferred_element_type=jnp.float32)
    adst = jnp.dot(h2, adw_ref[...], preferred_element_type=jnp.float32)
    asrc_ref[...] = asrc
    adst_ref[...] = adst
    a = asrc + adst
    wself2_ref[...] = jnp.exp(jnp.maximum(a, 0.2 * a))


_tc_mid = pl.pallas_call(
    _tc_mid_body,
    out_shape=[
        jax.ShapeDtypeStruct((N, D), jnp.float32),
        jax.ShapeDtypeStruct((N, 1), jnp.float32),
        jax.ShapeDtypeStruct((N, 1), jnp.float32),
        jax.ShapeDtypeStruct((N, 1), jnp.float32),
    ],
)


def _tc_post_body(acc_ref, den_ref, h_ref, wself_ref, b_ref, batch_ref,
                  out_ref):
    num = acc_ref[0, :N, :] + acc_ref[1, :N, :] + wself_ref[...] * h_ref[...]
    den = (den_ref[0:1, :N] + den_ref[1:2, :N]).reshape(N, 1) + \
        wself_ref[...] + 1e-16
    y = _selu(num / den + b_ref[...])
    gids = lax.broadcasted_iota(jnp.int32, (G, N), 0)
    onehot = (gids == batch_ref[...]).astype(jnp.float32)
    sums = jnp.dot(onehot, y, preferred_element_type=jnp.float32)
    cnts = jnp.sum(onehot, axis=1, keepdims=True)
    out_ref[...] = sums / jnp.clip(cnts, 1.0, None)


_tc_post = pl.pallas_call(
    _tc_post_body,
    out_shape=jax.ShapeDtypeStruct((G, D), jnp.float32),
)


def kernel(u, edge_index, batch, W1, att_src1, att_dst1, b1,
           W2, att_src2, att_dst2, b2):
    ei = jnp.asarray(edge_index, jnp.int32)
    src3 = ei[0]
    dst3 = ei[1]
    batch2 = jnp.asarray(batch, jnp.int32).reshape(1, N)
    z2 = jnp.zeros((NPAD, D), jnp.float32)
    z1 = jnp.zeros((NPAD,), jnp.float32)

    h1, asrc1, adst1, wself1 = _tc_pre(
        u, W1, att_src1.reshape(D, 1), att_dst1.reshape(D, 1))
    acc1, den1 = _sc_layer(src3, dst3, asrc1.reshape(N), adst1.reshape(N),
                           h1, z2, z1)
    h2, asrc2, adst2, wself2 = _tc_mid(
        acc1, den1.reshape(NC, NPAD), h1, wself1, b1.reshape(1, D), W2,
        att_src2.reshape(D, 1), att_dst2.reshape(D, 1))
    acc2, den2 = _sc_layer(src3, dst3, asrc2.reshape(N), adst2.reshape(N),
                           h2, z2, z1)
    return _tc_post(acc2, den2.reshape(NC, NPAD), h2, wself2,
                    b2.reshape(1, D), batch2)
